# baseline (speedup 1.0000x reference)
import functools

import jax
import jax.numpy as jnp
from jax import lax
from jax.experimental import pallas as pl
from jax.experimental.pallas import tpu as pltpu
from jax.experimental.pallas import tpu_sc as plsc

EMB_DIM = 64
_PARAMS = pltpu.CompilerParams(needs_layout_passes=False)


@functools.lru_cache(maxsize=None)
def _make_format(V: int):
    info = plsc.get_sparse_core_info()
    nw = info.num_cores * info.num_subcores
    nblk_full = V // 128
    tail = V - nblk_full * 128
    blk_per_w = (nblk_full + nw - 1) // nw
    mesh = plsc.VectorSubcoreMesh(core_axis_name="c", subcore_axis_name="s")

    @functools.partial(
        pl.kernel,
        mesh=mesh,
        compiler_params=_PARAMS,
        out_type=jax.ShapeDtypeStruct((V // 2, 128), jnp.float32),
        scratch_types=[
            pltpu.VMEM((EMB_DIM, 128), jnp.float32),
            pltpu.VMEM((EMB_DIM, 128), jnp.float32),
            pltpu.VMEM((EMB_DIM, EMB_DIM), jnp.float32),
            pltpu.VMEM((32, 128), jnp.float32),
        ],
    )
    def format_kernel(tt_hbm, t2_hbm, col_v, rows_v, tcol_v, trows_v):
        wid = lax.axis_index("s") * info.num_cores + lax.axis_index("c")
        iota16 = lax.iota(jnp.int32, 16)
        rsrc = [(16 * g + iota16) % EMB_DIM for g in range(8)]
        cadd = [(16 * g) // EMB_DIM for g in range(8)]

        def blk_body(t, carry):
            j = t * nw + wid

            @pl.when(j < nblk_full)
            def _():
                pltpu.sync_copy(tt_hbm.at[:, pl.ds(j * 128, 128)], col_v)

                def rbody(r, c):
                    for g in range(8):
                        cidx = lax.broadcast(2 * r + cadd[g], (16,))
                        vals = plsc.load_gather(col_v, [rsrc[g], cidx])
                        rows_v[r, pl.ds(16 * g, 16)] = vals
                    return c

                lax.fori_loop(0, EMB_DIM, rbody, 0)
                pltpu.sync_copy(rows_v, t2_hbm.at[pl.ds(j * 64, 64)])

            return carry

        lax.fori_loop(0, blk_per_w, blk_body, 0)

        @pl.when(jnp.logical_and(wid == 0, tail > 0))
        def _():
            pltpu.sync_copy(tt_hbm.at[:, pl.ds(nblk_full * 128, tail)], tcol_v)

            def rbody(r, c):
                for g in range(8):
                    cidx = lax.broadcast(2 * r + cadd[g], (16,))
                    vals = plsc.load_gather(tcol_v, [rsrc[g], cidx])
                    trows_v[r, pl.ds(16 * g, 16)] = vals
                return c

            lax.fori_loop(0, tail // 2, rbody, 0)
            pltpu.sync_copy(trows_v, t2_hbm.at[pl.ds(nblk_full * 64, tail // 2)])

    return format_kernel


@functools.lru_cache(maxsize=None)
def _make_gather(S: int, B: int, V2: int):
    info = plsc.get_sparse_core_info()
    nw = info.num_cores * info.num_subcores
    assert B == 128 * nw
    mesh = plsc.VectorSubcoreMesh(core_axis_name="c", subcore_axis_name="s")

    @functools.partial(
        pl.kernel,
        mesh=mesh,
        compiler_params=_PARAMS,
        out_type=jax.ShapeDtypeStruct((S, EMB_DIM, B), jnp.float32),
        scratch_types=[
            pltpu.VMEM((128,), jnp.int32),
            pltpu.VMEM((128,), jnp.int32),
            pltpu.VMEM((128,), jnp.int32),
            pltpu.VMEM((128, 128), jnp.float32),
            pltpu.VMEM((EMB_DIM, 128), jnp.float32),
            pltpu.SemaphoreType.DMA,
        ],
    )
    def gather_kernel(xt_hbm, t2_hbm, out_hbm, idx_v, idx2_v, base_v, rows_v,
                      tiles_v, sem):
        wid = lax.axis_index("s") * info.num_cores + lax.axis_index("c")
        boff = wid * 128
        iota16 = lax.iota(jnp.int32, 16)

        def body(s, carry):
            pltpu.sync_copy(xt_hbm.at[s, pl.ds(boff, 128)], idx_v)
            for g in range(8):
                v = idx_v[pl.ds(16 * g, 16)]
                idx2_v[pl.ds(16 * g, 16)] = v >> 1
                base_v[pl.ds(16 * g, 16)] = (v & 1) * EMB_DIM
            pltpu.async_copy(t2_hbm.at[idx2_v], rows_v, sem).wait()

            def dbody(d, c):
                for g in range(8):
                    col = base_v[pl.ds(16 * g, 16)] + d
                    vals = plsc.load_gather(rows_v, [iota16 + 16 * g, col])
                    tiles_v[d, pl.ds(16 * g, 16)] = vals
                return c

            lax.fori_loop(0, EMB_DIM, dbody, 0)
            pltpu.sync_copy(tiles_v, out_hbm.at[s, :, pl.ds(boff, 128)])
            return carry

        lax.fori_loop(0, S, body, 0)

    return gather_kernel


def kernel(x, table):
    b, s = x.shape
    v, d = table.shape
    t2 = _make_format(v)(table.T)
    out_t = _make_gather(s, b, v // 2)(x.T, t2)
    return out_t.transpose(2, 0, 1)


# speedup vs baseline: 1.5367x; 1.5367x over previous
import functools

import jax
import jax.numpy as jnp
from jax import lax
from jax.experimental import pallas as pl
from jax.experimental.pallas import tpu as pltpu
from jax.experimental.pallas import tpu_sc as plsc

EMB_DIM = 64
_PARAMS = pltpu.CompilerParams(needs_layout_passes=False)


@functools.lru_cache(maxsize=None)
def _make_format(V: int):
    info = plsc.get_sparse_core_info()
    nw = info.num_cores * info.num_subcores
    nblk = V // 128
    tail = V - nblk * 128
    npair = (nblk // nw + 1 + 1) // 2
    mesh = plsc.VectorSubcoreMesh(core_axis_name="c", subcore_axis_name="s")

    @functools.partial(
        pl.kernel,
        mesh=mesh,
        compiler_params=_PARAMS,
        out_type=jax.ShapeDtypeStruct((V // 2, 128), jnp.float32),
        scratch_types=[
            pltpu.VMEM((EMB_DIM, 128), jnp.float32),
            pltpu.VMEM((EMB_DIM, 128), jnp.float32),
            pltpu.VMEM((EMB_DIM, 128), jnp.float32),
            pltpu.VMEM((EMB_DIM, 128), jnp.float32),
            pltpu.VMEM((EMB_DIM, EMB_DIM), jnp.float32),
            pltpu.VMEM((32, 128), jnp.float32),
            pltpu.SemaphoreType.DMA,
            pltpu.SemaphoreType.DMA,
            pltpu.SemaphoreType.DMA,
            pltpu.SemaphoreType.DMA,
        ],
    )
    def format_kernel(tt_hbm, t2_hbm, col_a, col_b, rows_a, rows_b,
                      tcol_v, trows_v, lsem_a, lsem_b, ssem_a, ssem_b):
        wid = lax.axis_index("s") * info.num_cores + lax.axis_index("c")
        iota16 = lax.iota(jnp.int32, 16)
        rsrc = [(16 * g + iota16) % EMB_DIM for g in range(8)]
        cadd = [(16 * g) // EMB_DIM for g in range(8)]

        def blk_of(t):
            return t * nw + wid

        def active(t):
            return blk_of(t) < nblk

        def start_load(t, col, lsem):
            j = blk_of(t)
            pltpu.async_copy(tt_hbm.at[:, pl.ds(j * 128, 128)], col, lsem)

        def wait_load(t, col, lsem):
            j = blk_of(t)
            pltpu.make_async_copy(
                tt_hbm.at[:, pl.ds(j * 128, 128)], col, lsem).wait()

        def transpose(col, rows):
            def rbody(r, c):
                for g in range(8):
                    cidx = lax.broadcast(2 * r + cadd[g], (16,))
                    rows[r, pl.ds(16 * g, 16)] = plsc.load_gather(
                        col, [rsrc[g], cidx])
                return c

            lax.fori_loop(0, EMB_DIM, rbody, 0)

        def start_store(t, rows, ssem):
            j = blk_of(t)
            pltpu.async_copy(rows, t2_hbm.at[pl.ds(j * 64, 64)], ssem)

        def wait_store(t, rows, ssem):
            j = blk_of(t)
            pltpu.make_async_copy(rows, t2_hbm.at[pl.ds(j * 64, 64)],
                                  ssem).wait()

        start_load(0, col_a, lsem_a)

        def pair_body(p, carry):
            t0 = 2 * p
            t1 = t0 + 1

            @pl.when(active(t1))
            def _():
                start_load(t1, col_b, lsem_b)

            @pl.when(jnp.logical_and(p > 0, active(t0)))
            def _():
                wait_store(t0 - 2, rows_a, ssem_a)

            @pl.when(active(t0))
            def _():
                wait_load(t0, col_a, lsem_a)
                transpose(col_a, rows_a)
                start_store(t0, rows_a, ssem_a)

            @pl.when(jnp.logical_and(p + 1 < npair, active(t0 + 2)))
            def _():
                start_load(t0 + 2, col_a, lsem_a)

            @pl.when(jnp.logical_and(p > 0, active(t1)))
            def _():
                wait_store(t1 - 2, rows_b, ssem_b)

            @pl.when(active(t1))
            def _():
                wait_load(t1, col_b, lsem_b)
                transpose(col_b, rows_b)
                start_store(t1, rows_b, ssem_b)

            return carry

        lax.fori_loop(0, npair, pair_body, 0)

        nact = (nblk - wid + nw - 1) // nw
        last_even = ((nact - 1) // 2) * 2
        last_odd = ((nact - 2) // 2) * 2 + 1
        wait_store(last_even, rows_a, ssem_a)
        wait_store(last_odd, rows_b, ssem_b)

        @pl.when(jnp.logical_and(wid == 0, tail > 0))
        def _():
            pltpu.sync_copy(tt_hbm.at[:, pl.ds(nblk * 128, tail)], tcol_v)

            def rbody(r, c):
                for g in range(8):
                    cidx = lax.broadcast(2 * r + cadd[g], (16,))
                    trows_v[r, pl.ds(16 * g, 16)] = plsc.load_gather(
                        tcol_v, [rsrc[g], cidx])
                return c

            lax.fori_loop(0, tail // 2, rbody, 0)
            pltpu.sync_copy(trows_v, t2_hbm.at[pl.ds(nblk * 64, tail // 2)])

    return format_kernel


@functools.lru_cache(maxsize=None)
def _make_gather(S: int, B: int, V2: int):
    info = plsc.get_sparse_core_info()
    nw = info.num_cores * info.num_subcores
    assert B == 128 * nw and S % 2 == 0
    mesh = plsc.VectorSubcoreMesh(core_axis_name="c", subcore_axis_name="s")

    @functools.partial(
        pl.kernel,
        mesh=mesh,
        compiler_params=_PARAMS,
        out_type=jax.ShapeDtypeStruct((S, EMB_DIM, B), jnp.float32),
        scratch_types=[
            pltpu.VMEM((S, 128), jnp.int32),
            pltpu.VMEM((S, 128), jnp.int32),
            pltpu.VMEM((128, 128), jnp.float32),
            pltpu.VMEM((128, 128), jnp.float32),
            pltpu.VMEM((EMB_DIM, 128), jnp.float32),
            pltpu.VMEM((EMB_DIM, 128), jnp.float32),
            pltpu.SemaphoreType.DMA,
            pltpu.SemaphoreType.DMA,
            pltpu.SemaphoreType.DMA,
            pltpu.SemaphoreType.DMA,
        ],
    )
    def gather_kernel(xt_hbm, t2_hbm, out_hbm, idx2_v, base_v, rows_a, rows_b,
                      tiles_a, tiles_b, gsem_a, gsem_b, ssem_a, ssem_b):
        wid = lax.axis_index("s") * info.num_cores + lax.axis_index("c")
        boff = wid * 128
        iota16 = lax.iota(jnp.int32, 16)

        pltpu.sync_copy(xt_hbm.at[:, pl.ds(boff, 128)], idx2_v)

        def prep_body(s, carry):
            for g in range(8):
                v = idx2_v[s, pl.ds(16 * g, 16)]
                idx2_v[s, pl.ds(16 * g, 16)] = v >> 1
                base_v[s, pl.ds(16 * g, 16)] = (v & 1) * EMB_DIM
            return carry

        lax.fori_loop(0, S, prep_body, 0)

        def start_gather(s, rows, gsem):
            pltpu.async_copy(t2_hbm.at[idx2_v.at[s]], rows, gsem)

        def wait_gather(s, rows, gsem):
            pltpu.make_async_copy(t2_hbm.at[idx2_v.at[s]], rows, gsem).wait()

        def transpose(s, rows, tiles):
            bases = tuple(base_v[s, pl.ds(16 * g, 16)] for g in range(8))

            def dbody(d, carry):
                for g in range(8):
                    vals = plsc.load_gather(rows, [iota16 + 16 * g,
                                                   carry[g] + d])
                    tiles[d, pl.ds(16 * g, 16)] = vals
                return carry

            lax.fori_loop(0, EMB_DIM, dbody, bases)

        def start_store(s, tiles, ssem):
            pltpu.async_copy(tiles, out_hbm.at[s, :, pl.ds(boff, 128)], ssem)

        def wait_store(s, tiles, ssem):
            pltpu.make_async_copy(tiles, out_hbm.at[s, :, pl.ds(boff, 128)],
                                  ssem).wait()

        start_gather(0, rows_a, gsem_a)

        def pair_body(p, carry):
            s0 = 2 * p
            s1 = s0 + 1

            start_gather(s1, rows_b, gsem_b)

            @pl.when(p > 0)
            def _():
                wait_store(s0 - 2, tiles_a, ssem_a)

            wait_gather(s0, rows_a, gsem_a)
            transpose(s0, rows_a, tiles_a)
            start_store(s0, tiles_a, ssem_a)

            @pl.when(p + 1 < S // 2)
            def _():
                start_gather(s0 + 2, rows_a, gsem_a)

            @pl.when(p > 0)
            def _():
                wait_store(s1 - 2, tiles_b, ssem_b)

            wait_gather(s1, rows_b, gsem_b)
            transpose(s1, rows_b, tiles_b)
            start_store(s1, tiles_b, ssem_b)
            return carry

        lax.fori_loop(0, S // 2, pair_body, 0)
        wait_store(S - 2, tiles_a, ssem_a)
        wait_store(S - 1, tiles_b, ssem_b)

    return gather_kernel


def kernel(x, table):
    b, s = x.shape
    v, d = table.shape
    t2 = _make_format(v)(table.T)
    out_t = _make_gather(s, b, v // 2)(x.T, t2)
    return out_t.transpose(2, 0, 1)


# speedup vs baseline: 4.3604x; 2.8375x over previous
import functools

import jax
import jax.numpy as jnp
from jax import lax
from jax.experimental import pallas as pl
from jax.experimental.pallas import tpu as pltpu
from jax.experimental.pallas import tpu_sc as plsc

EMB_DIM = 64
_PARAMS = pltpu.CompilerParams(needs_layout_passes=False)


@functools.lru_cache(maxsize=None)
def _make_format(V: int):
    info = plsc.get_sparse_core_info()
    nw = info.num_cores * info.num_subcores
    nblk = V // 128
    tail = V - nblk * 128
    npair = (nblk // nw + 1 + 1) // 2
    mesh = plsc.VectorSubcoreMesh(core_axis_name="c", subcore_axis_name="s")

    @functools.partial(
        pl.kernel,
        mesh=mesh,
        compiler_params=_PARAMS,
        out_type=jax.ShapeDtypeStruct((V // 2, 128), jnp.float32),
        scratch_types=[
            pltpu.VMEM((EMB_DIM, 128), jnp.float32),
            pltpu.VMEM((EMB_DIM, 128), jnp.float32),
            pltpu.VMEM((EMB_DIM, 128), jnp.float32),
            pltpu.VMEM((EMB_DIM, 128), jnp.float32),
            pltpu.VMEM((EMB_DIM, EMB_DIM), jnp.float32),
            pltpu.VMEM((32, 128), jnp.float32),
            pltpu.SemaphoreType.DMA,
            pltpu.SemaphoreType.DMA,
            pltpu.SemaphoreType.DMA,
            pltpu.SemaphoreType.DMA,
        ],
    )
    def format_kernel(tt_hbm, t2_hbm, col_a, col_b, rows_a, rows_b,
                      tcol_v, trows_v, lsem_a, lsem_b, ssem_a, ssem_b):
        wid = lax.axis_index("s") * info.num_cores + lax.axis_index("c")
        iota16 = lax.iota(jnp.int32, 16)
        rsrc = [(16 * g + iota16) % EMB_DIM for g in range(8)]
        cadd = [(16 * g) // EMB_DIM for g in range(8)]
        ccs = [iota16 + 16 * g for g in range(8)]

        def blk_of(t):
            return t * nw + wid

        def active(t):
            return blk_of(t) < nblk

        def start_load(t, col, lsem):
            j = blk_of(t)
            pltpu.async_copy(tt_hbm.at[:, pl.ds(j * 128, 128)], col, lsem)

        def wait_load(t, col, lsem):
            j = blk_of(t)
            pltpu.make_async_copy(
                tt_hbm.at[:, pl.ds(j * 128, 128)], col, lsem).wait()

        def transpose(col, rows):
            def pbody(pp, c):
                r16 = (pp + iota16) & (EMB_DIM - 1)
                col2 = 2 * r16
                for g in range(8):
                    vals = plsc.load_gather(col, [rsrc[g], col2 + cadd[g]])
                    plsc.store_scatter(rows, [r16, ccs[g]], vals)
                return c

            lax.fori_loop(0, EMB_DIM, pbody, 0)

        def start_store(t, rows, ssem):
            j = blk_of(t)
            pltpu.async_copy(rows, t2_hbm.at[pl.ds(j * 64, 64)], ssem)

        def wait_store(t, rows, ssem):
            j = blk_of(t)
            pltpu.make_async_copy(rows, t2_hbm.at[pl.ds(j * 64, 64)],
                                  ssem).wait()

        start_load(0, col_a, lsem_a)

        def pair_body(p, carry):
            t0 = 2 * p
            t1 = t0 + 1

            @pl.when(active(t1))
            def _():
                start_load(t1, col_b, lsem_b)

            @pl.when(jnp.logical_and(p > 0, active(t0)))
            def _():
                wait_store(t0 - 2, rows_a, ssem_a)

            @pl.when(active(t0))
            def _():
                wait_load(t0, col_a, lsem_a)
                transpose(col_a, rows_a)
                start_store(t0, rows_a, ssem_a)

            @pl.when(jnp.logical_and(p + 1 < npair, active(t0 + 2)))
            def _():
                start_load(t0 + 2, col_a, lsem_a)

            @pl.when(jnp.logical_and(p > 0, active(t1)))
            def _():
                wait_store(t1 - 2, rows_b, ssem_b)

            @pl.when(active(t1))
            def _():
                wait_load(t1, col_b, lsem_b)
                transpose(col_b, rows_b)
                start_store(t1, rows_b, ssem_b)

            return carry

        lax.fori_loop(0, npair, pair_body, 0)

        nact = (nblk - wid + nw - 1) // nw
        last_even = ((nact - 1) // 2) * 2
        last_odd = ((nact - 2) // 2) * 2 + 1
        wait_store(last_even, rows_a, ssem_a)
        wait_store(last_odd, rows_b, ssem_b)

        @pl.when(jnp.logical_and(wid == 0, tail > 0))
        def _():
            pltpu.sync_copy(tt_hbm.at[:, pl.ds(nblk * 128, tail)], tcol_v)

            def pbody(pp, c):
                r16 = (pp + iota16) & (tail // 2 - 1)
                col2 = 2 * r16
                for g in range(8):
                    vals = plsc.load_gather(tcol_v, [rsrc[g], col2 + cadd[g]])
                    plsc.store_scatter(trows_v, [r16, ccs[g]], vals)
                return c

            lax.fori_loop(0, tail // 2, pbody, 0)
            pltpu.sync_copy(trows_v, t2_hbm.at[pl.ds(nblk * 64, tail // 2)])

    return format_kernel


@functools.lru_cache(maxsize=None)
def _make_gather(S: int, B: int, V2: int):
    info = plsc.get_sparse_core_info()
    nw = info.num_cores * info.num_subcores
    assert B == 128 * nw and S % 2 == 0
    mesh = plsc.VectorSubcoreMesh(core_axis_name="c", subcore_axis_name="s")

    @functools.partial(
        pl.kernel,
        mesh=mesh,
        compiler_params=_PARAMS,
        out_type=jax.ShapeDtypeStruct((S, EMB_DIM, B), jnp.float32),
        scratch_types=[
            pltpu.VMEM((S, 128), jnp.int32),
            pltpu.VMEM((S, 128), jnp.int32),
            pltpu.VMEM((128, 128), jnp.float32),
            pltpu.VMEM((128, 128), jnp.float32),
            pltpu.VMEM((EMB_DIM, 128), jnp.float32),
            pltpu.VMEM((EMB_DIM, 128), jnp.float32),
            pltpu.SemaphoreType.DMA,
            pltpu.SemaphoreType.DMA,
            pltpu.SemaphoreType.DMA,
            pltpu.SemaphoreType.DMA,
        ],
    )
    def gather_kernel(xt_hbm, t2_hbm, out_hbm, idx2_v, base_v, rows_a, rows_b,
                      tiles_a, tiles_b, gsem_a, gsem_b, ssem_a, ssem_b):
        wid = lax.axis_index("s") * info.num_cores + lax.axis_index("c")
        boff = wid * 128
        iota16 = lax.iota(jnp.int32, 16)

        pltpu.sync_copy(xt_hbm.at[:, pl.ds(boff, 128)], idx2_v)

        def prep_body(s, carry):
            for g in range(8):
                v = idx2_v[s, pl.ds(16 * g, 16)]
                idx2_v[s, pl.ds(16 * g, 16)] = v >> 1
                base_v[s, pl.ds(16 * g, 16)] = (v & 1) * EMB_DIM
            return carry

        lax.fori_loop(0, S, prep_body, 0)

        def start_gather(s, rows, gsem):
            pltpu.async_copy(t2_hbm.at[idx2_v.at[s]], rows, gsem)

        def wait_gather(s, rows, gsem):
            pltpu.make_async_copy(t2_hbm.at[idx2_v.at[s]], rows, gsem).wait()

        def transpose(s, rows, tiles):
            bases = tuple(base_v[s, pl.ds(16 * g, 16)] for g in range(8))
            bbs = tuple(iota16 + 16 * g for g in range(8))

            def dbody(dd, carry):
                d16 = (dd + iota16) & (EMB_DIM - 1)
                for g in range(8):
                    vals = plsc.load_gather(rows, [bbs[g], carry[g] + d16])
                    plsc.store_scatter(tiles, [d16, bbs[g]], vals)
                return carry

            lax.fori_loop(0, EMB_DIM, dbody, bases)

        def start_store(s, tiles, ssem):
            pltpu.async_copy(tiles, out_hbm.at[s, :, pl.ds(boff, 128)], ssem)

        def wait_store(s, tiles, ssem):
            pltpu.make_async_copy(tiles, out_hbm.at[s, :, pl.ds(boff, 128)],
                                  ssem).wait()

        start_gather(0, rows_a, gsem_a)

        def pair_body(p, carry):
            s0 = 2 * p
            s1 = s0 + 1

            start_gather(s1, rows_b, gsem_b)

            @pl.when(p > 0)
            def _():
                wait_store(s0 - 2, tiles_a, ssem_a)

            wait_gather(s0, rows_a, gsem_a)
            transpose(s0, rows_a, tiles_a)
            start_store(s0, tiles_a, ssem_a)

            @pl.when(p + 1 < S // 2)
            def _():
                start_gather(s0 + 2, rows_a, gsem_a)

            @pl.when(p > 0)
            def _():
                wait_store(s1 - 2, tiles_b, ssem_b)

            wait_gather(s1, rows_b, gsem_b)
            transpose(s1, rows_b, tiles_b)
            start_store(s1, tiles_b, ssem_b)
            return carry

        lax.fori_loop(0, S // 2, pair_body, 0)
        wait_store(S - 2, tiles_a, ssem_a)
        wait_store(S - 1, tiles_b, ssem_b)

    return gather_kernel


def kernel(x, table):
    b, s = x.shape
    v, d = table.shape
    t2 = _make_format(v)(table.T)
    out_t = _make_gather(s, b, v // 2)(x.T, t2)
    return out_t.transpose(2, 0, 1)


# speedup vs baseline: 4.3621x; 1.0004x over previous
import functools

import jax
import jax.numpy as jnp
from jax import lax
from jax.experimental import pallas as pl
from jax.experimental.pallas import tpu as pltpu
from jax.experimental.pallas import tpu_sc as plsc

EMB_DIM = 64
_PARAMS = pltpu.CompilerParams(needs_layout_passes=False)


@functools.lru_cache(maxsize=None)
def _make_format(V: int):
    info = plsc.get_sparse_core_info()
    nw = info.num_cores * info.num_subcores
    nblk = V // 128
    tail = V - nblk * 128
    nslot = nblk // nw + 1
    nq = (nslot + 3) // 4
    mesh = plsc.VectorSubcoreMesh(core_axis_name="c", subcore_axis_name="s")

    @functools.partial(
        pl.kernel,
        mesh=mesh,
        compiler_params=_PARAMS,
        out_type=jax.ShapeDtypeStruct((V // 2, 128), jnp.float32),
        scratch_types=[
            pltpu.VMEM((4, EMB_DIM, 128), jnp.float32),
            pltpu.VMEM((4, EMB_DIM, 128), jnp.float32),
            pltpu.VMEM((EMB_DIM, EMB_DIM), jnp.float32),
            pltpu.VMEM((32, 128), jnp.float32),
            [pltpu.SemaphoreType.DMA] * 4,
            [pltpu.SemaphoreType.DMA] * 4,
        ],
    )
    def format_kernel(tt_hbm, t2_hbm, col_v, rows_v, tcol_v, trows_v,
                      lsems, ssems):
        wid = lax.axis_index("s") * info.num_cores + lax.axis_index("c")
        iota16 = lax.iota(jnp.int32, 16)
        rsrc = [(16 * g + iota16) % EMB_DIM for g in range(8)]
        cadd = [(16 * g) // EMB_DIM for g in range(8)]
        ccs = [iota16 + 16 * g for g in range(8)]

        def blk_of(t):
            return t * nw + wid

        def active(t):
            return blk_of(t) < nblk

        def start_load(t, ln):
            j = blk_of(t)
            pltpu.async_copy(tt_hbm.at[:, pl.ds(j * 128, 128)],
                             col_v.at[ln], lsems[ln])

        def wait_load(ln):
            pltpu.make_async_copy(tt_hbm.at[:, pl.ds(0, 128)],
                                  col_v.at[ln], lsems[ln]).wait()

        def transpose(col, rows):
            def pbody(pp, c):
                r16 = (pp + iota16) & (EMB_DIM - 1)
                col2 = 2 * r16
                for g in range(8):
                    vals = plsc.load_gather(col, [rsrc[g], col2 + cadd[g]])
                    plsc.store_scatter(rows, [r16, ccs[g]], vals)
                return c

            lax.fori_loop(0, EMB_DIM, pbody, 0)

        def start_store(t, ln):
            j = blk_of(t)
            pltpu.async_copy(rows_v.at[ln], t2_hbm.at[pl.ds(j * 64, 64)],
                             ssems[ln])

        def wait_store(ln):
            pltpu.make_async_copy(rows_v.at[ln], t2_hbm.at[pl.ds(0, 64)],
                                  ssems[ln]).wait()

        for ln in range(4):
            start_load(ln, ln)

        def quad_body(q, carry):
            for ln in range(4):
                t = 4 * q + ln

                @pl.when(active(t))
                def _(t=t, ln=ln):
                    @pl.when(q > 0)
                    def _():
                        wait_store(ln)

                    wait_load(ln)
                    transpose(col_v.at[ln], rows_v.at[ln])
                    start_store(t, ln)

                @pl.when(jnp.logical_and(q + 1 < nq, active(t + 4)))
                def _(t=t, ln=ln):
                    start_load(t + 4, ln)

            return carry

        lax.fori_loop(0, nq, quad_body, 0)
        for ln in range(4):
            wait_store(ln)

        @pl.when(jnp.logical_and(wid == 0, tail > 0))
        def _():
            pltpu.sync_copy(tt_hbm.at[:, pl.ds(nblk * 128, tail)], tcol_v)

            def pbody(pp, c):
                r16 = (pp + iota16) & (tail // 2 - 1)
                col2 = 2 * r16
                for g in range(8):
                    vals = plsc.load_gather(tcol_v, [rsrc[g], col2 + cadd[g]])
                    plsc.store_scatter(trows_v, [r16, ccs[g]], vals)
                return c

            lax.fori_loop(0, tail // 2, pbody, 0)
            pltpu.sync_copy(trows_v, t2_hbm.at[pl.ds(nblk * 64, tail // 2)])

    return format_kernel


@functools.lru_cache(maxsize=None)
def _make_gather(S: int, B: int, V2: int):
    info = plsc.get_sparse_core_info()
    nw = info.num_cores * info.num_subcores
    assert B == 128 * nw and S % 4 == 0
    nq = S // 4
    mesh = plsc.VectorSubcoreMesh(core_axis_name="c", subcore_axis_name="s")

    @functools.partial(
        pl.kernel,
        mesh=mesh,
        compiler_params=_PARAMS,
        out_type=jax.ShapeDtypeStruct((S, EMB_DIM, B), jnp.float32),
        scratch_types=[
            pltpu.VMEM((S, 128), jnp.int32),
            pltpu.VMEM((4, 128), jnp.int32),
            pltpu.VMEM((4, 128, 128), jnp.float32),
            pltpu.VMEM((4, EMB_DIM, 128), jnp.float32),
            [pltpu.SemaphoreType.DMA] * 4,
            [pltpu.SemaphoreType.DMA] * 4,
        ],
    )
    def gather_kernel(xt_hbm, t2_hbm, out_hbm, idx_v, idx2_v, rows_v,
                      tiles_v, gsems, ssems):
        wid = lax.axis_index("s") * info.num_cores + lax.axis_index("c")
        boff = wid * 128
        iota16 = lax.iota(jnp.int32, 16)
        bbs = tuple(iota16 + 16 * g for g in range(8))

        pltpu.sync_copy(xt_hbm.at[:, pl.ds(boff, 128)], idx_v)

        def start_gather(s, ln):
            for g in range(8):
                idx2_v[ln, pl.ds(16 * g, 16)] = idx_v[s, pl.ds(16 * g, 16)] >> 1
            pltpu.async_copy(t2_hbm.at[idx2_v.at[ln]], rows_v.at[ln],
                             gsems[ln])

        def wait_gather(ln):
            pltpu.make_async_copy(t2_hbm.at[idx2_v.at[ln]], rows_v.at[ln],
                                  gsems[ln]).wait()

        def transpose(s, ln):
            rows = rows_v.at[ln]
            tiles = tiles_v.at[ln]
            bases = tuple(
                (idx_v[s, pl.ds(16 * g, 16)] & 1) * EMB_DIM for g in range(8))

            def dbody(dd, carry):
                d16 = (dd + iota16) & (EMB_DIM - 1)
                for g in range(8):
                    vals = plsc.load_gather(rows, [bbs[g], carry[g] + d16])
                    plsc.store_scatter(tiles, [d16, bbs[g]], vals)
                return carry

            lax.fori_loop(0, EMB_DIM, dbody, bases)

        def start_store(s, ln):
            pltpu.async_copy(tiles_v.at[ln],
                             out_hbm.at[s, :, pl.ds(boff, 128)], ssems[ln])

        def wait_store(ln):
            pltpu.make_async_copy(tiles_v.at[ln],
                                  out_hbm.at[0, :, pl.ds(boff, 128)],
                                  ssems[ln]).wait()

        for ln in range(4):
            start_gather(ln, ln)

        def quad_body(q, carry):
            for ln in range(4):
                s = 4 * q + ln

                @pl.when(q > 0)
                def _(ln=ln):
                    wait_store(ln)

                wait_gather(ln)
                transpose(s, ln)

                @pl.when(q + 1 < nq)
                def _(s=s, ln=ln):
                    start_gather(s + 4, ln)

                start_store(s, ln)
            return carry

        lax.fori_loop(0, nq, quad_body, 0)
        for ln in range(4):
            wait_store(ln)

    return gather_kernel


def kernel(x, table):
    b, s = x.shape
    v, d = table.shape
    t2 = _make_format(v)(table.T)
    out_t = _make_gather(s, b, v // 2)(x.T, t2)
    return out_t.transpose(2, 0, 1)
